# baseline (speedup 1.0000x reference)
import functools

import jax
import jax.numpy as jnp
from jax import lax
from jax.experimental import pallas as pl
from jax.experimental.pallas import tpu as pltpu
from jax.experimental.pallas import tpu_sc as plsc

_THRESHOLD = 0.35
_VAR0, _VAR1 = 0.1, 0.2
_NEG_RATIO = 3
_B, _P, _NO = 16, 25600, 32
_PS, _PL = 200, 128


def _match_body(pt_ref, tgt_ref, loc_ref, conf_ref, ce_ref, part_ref):
    cx, cy, w, h = pt_ref[0], pt_ref[1], pt_ref[2], pt_ref[3]
    x0, y0 = cx - 0.5 * w, cy - 0.5 * h
    x1, y1 = cx + 0.5 * w, cy + 0.5 * h
    area_b = w * h
    pid = (jax.lax.broadcasted_iota(jnp.int32, (_PS, _PL), 0) * _PL
           + jax.lax.broadcasted_iota(jnp.int32, (_PS, _PL), 1))

    bto = jnp.full((_PS, _PL), -1.0, jnp.float32)
    bti = jnp.zeros((_PS, _PL), jnp.int32)
    f_idx = jnp.full((_PS, _PL), -1, jnp.int32)

    for j in range(_NO):
        tx0 = tgt_ref[0, j, 0]
        ty0 = tgt_ref[0, j, 1]
        tx1 = tgt_ref[0, j, 2]
        ty1 = tgt_ref[0, j, 3]
        iw = jnp.maximum(jnp.minimum(tx1, x1) - jnp.maximum(tx0, x0), 0.0)
        ih = jnp.maximum(jnp.minimum(ty1, y1) - jnp.maximum(ty0, y0), 0.0)
        inter = iw * ih
        aa = (tx1 - tx0) * (ty1 - ty0)
        ov = inter / (aa + area_b - inter)
        better = ov > bto
        bti = jnp.where(better, j, bti)
        bto = jnp.where(better, ov, bto)
        bpo = jnp.max(ov)
        bpi = jnp.min(jnp.where(ov == bpo, pid, _P))
        f_idx = jnp.where(pid == bpi, j, f_idx)

    forced = f_idx >= 0
    bti = jnp.where(forced, f_idx, bti)
    bto = jnp.where(forced, 2.0, bto)
    pos = bto >= _THRESHOLD
    posf = pos.astype(jnp.float32)

    msx = jnp.zeros((_PS, _PL), jnp.float32)
    mdx = jnp.zeros((_PS, _PL), jnp.float32)
    msy = jnp.zeros((_PS, _PL), jnp.float32)
    mdy = jnp.zeros((_PS, _PL), jnp.float32)
    for j in range(_NO):
        eq = bti == j
        msx = jnp.where(eq, tgt_ref[0, j, 0] + tgt_ref[0, j, 2], msx)
        mdx = jnp.where(eq, tgt_ref[0, j, 2] - tgt_ref[0, j, 0], mdx)
        msy = jnp.where(eq, tgt_ref[0, j, 1] + tgt_ref[0, j, 3], msy)
        mdy = jnp.where(eq, tgt_ref[0, j, 3] - tgt_ref[0, j, 1], mdy)

    inv_w, inv_h = 1.0 / w, 1.0 / h
    g_cx = (0.5 * msx - cx) * ((1.0 / _VAR0) * inv_w)
    g_cy = (0.5 * msy - cy) * ((1.0 / _VAR0) * inv_h)
    g_w = jnp.log(mdx * inv_w) * (1.0 / _VAR1)
    g_h = jnp.log(mdy * inv_h) * (1.0 / _VAR1)

    def _sl1(d):
        ad = jnp.abs(d)
        return jnp.where(ad < 1.0, 0.5 * d * d, ad - 0.5)

    sl1 = (_sl1(loc_ref[0, 0] - g_cx) + _sl1(loc_ref[0, 1] - g_cy)
           + _sl1(loc_ref[0, 2] - g_w) + _sl1(loc_ref[0, 3] - g_h))
    sl1_sum = jnp.sum(sl1 * posf)
    npos = jnp.sum(posf)

    c0, c1 = conf_ref[0, 0], conf_ref[0, 1]
    dng = jnp.where(pos, c0 - c1, c1 - c0)
    ce = jnp.maximum(dng, 0.0) + jnp.log1p(jnp.exp(-jnp.abs(dng)))
    pce = jnp.sum(ce * posf)

    ce_ref[...] = jnp.where(pos, -1.0, ce)[None]

    li = jax.lax.broadcasted_iota(jnp.int32, (1, 1, 128), 2)
    part_ref[...] = (jnp.where(li == 0, sl1_sum, 0.0)
                     + jnp.where(li == 1, npos, 0.0)
                     + jnp.where(li == 2, pce, 0.0))


def _select_body(ce_ref, part_ref, out_loc_ref, out_conf_ref):
    ce = ce_ref[...]
    part = part_ref[...]
    sl1 = part[:, :, 0:1]
    nposf = part[:, :, 1:2]
    pce = part[:, :, 2:3]

    s_total = jnp.sum(nposf)
    k = jnp.minimum(jnp.minimum(_NEG_RATIO * nposf, float(_P - 1)),
                    float(_P) - nposf)

    def _rowsum(x):
        return jnp.sum(jnp.sum(x, axis=1, keepdims=True), axis=2,
                       keepdims=True)

    ci = jax.lax.bitcast_convert_type(ce, jnp.int32)
    lo = jnp.zeros((_B, 1, 1), jnp.int32)
    hi = jnp.max(jnp.max(ci, axis=1, keepdims=True), axis=2,
                 keepdims=True) + 1

    def body(_, carry):
        lo, hi = carry
        mid = lo + jax.lax.div(hi - lo, 2)
        cnt = _rowsum(jnp.where(ci >= mid, 1.0, 0.0))
        ok = cnt >= k
        return jnp.where(ok, mid, lo), jnp.where(ok, hi, mid)

    lo, hi = jax.lax.fori_loop(0, 31, body, (lo, hi))
    t = lo
    tf = jax.lax.bitcast_convert_type(t, jnp.float32)
    gt = ci > t
    cnt_gt = _rowsum(jnp.where(gt, 1.0, 0.0))
    sum_gt = _rowsum(jnp.where(gt, ce, 0.0))
    neg_sum = sum_gt + (k - cnt_gt) * tf

    total_ce = jnp.sum(pce) + jnp.sum(neg_sum)
    total_sel = s_total + jnp.sum(k)
    out_loc_ref[...] = jnp.reshape(jnp.sum(sl1) / (4.0 * s_total) / s_total,
                                   (1, 1))
    out_conf_ref[...] = jnp.reshape(total_ce / total_sel / s_total, (1, 1))


_L = 16
_NB1 = 512
_NB2 = 1024


def _sc_select_body(ce_hbm, part_hbm, out_hbm, row_v, part_v, cnt1, sum1,
                    cnt2, sum2, cc1, cs1, cc2, cs2, stat_v, out_v, shared):
    r = lax.axis_index("s")
    li = lax.iota(jnp.int32, _L)
    ones = jnp.full((_L,), 1.0, jnp.float32)
    zeros = jnp.zeros((_L,), jnp.float32)

    pltpu.sync_copy(ce_hbm.at[r], row_v)
    pltpu.sync_copy(part_hbm.at[r], part_v)

    def _zero1(i, _):
        cnt1[pl.ds(i * _L, _L)] = zeros
        sum1[pl.ds(i * _L, _L)] = zeros
        return 0

    def _zero2(i, _):
        cnt2[pl.ds(i * _L, _L)] = zeros
        sum2[pl.ds(i * _L, _L)] = zeros
        return 0

    lax.fori_loop(0, _L * _NB1 // _L, _zero1, 0)
    lax.fori_loop(0, _L * _NB2 // _L, _zero2, 0)

    pv = part_v[pl.ds(0, _L)]
    sl1 = pv[0]
    npos = pv[1]
    pce = pv[2]
    k = jnp.minimum(jnp.minimum(_NEG_RATIO * npos, float(_P - 1)),
                    float(_P) - npos)

    def _hist1(i, _):
        v = row_v[pl.ds(i * _L, _L)]
        bits = lax.bitcast_convert_type(v, jnp.int32)
        msk = v >= 0.0
        b1 = jnp.clip(lax.shift_right_arithmetic(bits, 22), 0, _NB1 - 1)
        idx = li * _NB1 + b1
        plsc.addupdate_scatter(cnt1, [idx], ones, mask=msk)
        plsc.addupdate_scatter(sum1, [idx], v, mask=msk)
        return 0

    lax.fori_loop(0, _P // _L, _hist1, 0)

    def _compact(tab, out, nb):
        def body(i, _):
            acc = tab[pl.ds(i * _L, _L)]
            for l in range(1, _L):
                acc = acc + tab[pl.ds(l * nb + i * _L, _L)]
            out[pl.ds(i * _L, _L)] = acc
            return 0

        lax.fori_loop(0, nb // _L, body, 0)

    _compact(cnt1, cc1, _NB1)
    _compact(sum1, cs1, _NB1)

    def _scan(cnt_tab, sum_tab, nb, kk):
        def cond(c):
            b, rc, rs, done = c
            return jnp.logical_and(jnp.logical_not(done), b >= 0)

        def body(c):
            b, rc, rs, done = c
            cb = cnt_tab[pl.ds(b, _L)][0]
            sb = sum_tab[pl.ds(b, _L)][0]
            hit = rc + cb >= kk
            rc = jnp.where(hit, rc, rc + cb)
            rs = jnp.where(hit, rs, rs + sb)
            b = jnp.where(hit, b, b - 1)
            return b, rc, rs, hit

        b, rc, rs, _ = lax.while_loop(
            cond, body, (jnp.int32(nb - 1), 0.0, 0.0, False))
        return b, rc, rs

    b1s, cnt_gt1, sum_gt1 = _scan(cc1, cs1, _NB1, k)
    k2 = k - cnt_gt1

    def _hist2(i, _):
        v = row_v[pl.ds(i * _L, _L)]
        bits = lax.bitcast_convert_type(v, jnp.int32)
        b1 = jnp.clip(lax.shift_right_arithmetic(bits, 22), 0, _NB1 - 1)
        msk = jnp.logical_and(v >= 0.0, b1 == b1s)
        b2 = jnp.bitwise_and(lax.shift_right_arithmetic(bits, 12), _NB2 - 1)
        idx = li * _NB2 + b2
        plsc.addupdate_scatter(cnt2, [idx], ones, mask=msk)
        plsc.addupdate_scatter(sum2, [idx], v, mask=msk)
        return 0

    lax.fori_loop(0, _P // _L, _hist2, 0)

    _compact(cnt2, cc2, _NB2)
    _compact(sum2, cs2, _NB2)

    b2s, cnt_gt2, sum_gt2 = _scan(cc2, cs2, _NB2, k2)
    cnt_at_v = cc2[pl.ds(b2s, _L)]
    sum_at_v = cs2[pl.ds(b2s, _L)]
    mean_at = (sum_at_v / cnt_at_v)[0]
    neg_sum = sum_gt1 + sum_gt2 + (k2 - cnt_gt2) * mean_at

    stat_v[...] = (jnp.where(li == 0, neg_sum, 0.0)
                   + jnp.where(li == 1, k, 0.0)
                   + jnp.where(li == 2, npos, 0.0)
                   + jnp.where(li == 3, sl1, 0.0)
                   + jnp.where(li == 4, pce, 0.0))
    pltpu.sync_copy(stat_v, shared.at[pl.ds(r * _L, _L)])
    plsc.subcore_barrier()

    @pl.when(r == 0)
    def _combine():
        def _acc(i, acc):
            pltpu.sync_copy(shared.at[pl.ds(i * _L, _L)], stat_v)
            return acc + stat_v[...]

        acc = lax.fori_loop(0, _B, _acc, zeros)
        neg_tot = acc[0]
        k_tot = acc[1]
        s_tot = acc[2]
        sl1_tot = acc[3]
        pce_tot = acc[4]
        num_v = (jnp.where(li == 0, sl1_tot, 0.0)
                 + jnp.where(li == 1, pce_tot + neg_tot, 0.0))
        den_v = (jnp.where(li == 0, 4.0 * s_tot * s_tot, 1.0)
                 + jnp.where(li == 1, (s_tot + k_tot) * s_tot - 1.0, 0.0))
        out_v[...] = num_v / den_v
        pltpu.sync_copy(out_v, out_hbm)


def _sc_select(ce16, part2):
    mesh = plsc.VectorSubcoreMesh(core_axis_name="c", subcore_axis_name="s",
                                  num_cores=1)
    f = functools.partial(
        pl.kernel, mesh=mesh,
        compiler_params=pltpu.CompilerParams(needs_layout_passes=False),
        out_type=jax.ShapeDtypeStruct((_L,), jnp.float32),
        scratch_types=[
            pltpu.VMEM((_P,), jnp.float32),
            pltpu.VMEM((128,), jnp.float32),
            pltpu.VMEM((_L * _NB1,), jnp.float32),
            pltpu.VMEM((_L * _NB1,), jnp.float32),
            pltpu.VMEM((_L * _NB2,), jnp.float32),
            pltpu.VMEM((_L * _NB2,), jnp.float32),
            pltpu.VMEM((_NB1 + _L,), jnp.float32),
            pltpu.VMEM((_NB1 + _L,), jnp.float32),
            pltpu.VMEM((_NB2 + _L,), jnp.float32),
            pltpu.VMEM((_NB2 + _L,), jnp.float32),
            pltpu.VMEM((_L,), jnp.float32),
            pltpu.VMEM((_L,), jnp.float32),
            pltpu.VMEM_SHARED((_B * _L,), jnp.float32),
        ],
    )(_sc_select_body)
    return f(ce16, part2)


def kernel(loc_data, conf_data, priors, targets):
    pt = priors.T.reshape(4, _PS, _PL)
    loc_t = jnp.transpose(loc_data, (0, 2, 1)).reshape(_B, 4, _PS, _PL)
    conf_t = jnp.transpose(conf_data, (0, 2, 1)).reshape(_B, 2, _PS, _PL)

    ce, part = pl.pallas_call(
        _match_body,
        grid=(_B,),
        in_specs=[
            pl.BlockSpec((4, _PS, _PL), lambda i: (0, 0, 0)),
            pl.BlockSpec((1, _NO, 5), lambda i: (i, 0, 0),
                         memory_space=pltpu.SMEM),
            pl.BlockSpec((1, 4, _PS, _PL), lambda i: (i, 0, 0, 0)),
            pl.BlockSpec((1, 2, _PS, _PL), lambda i: (i, 0, 0, 0)),
        ],
        out_specs=[
            pl.BlockSpec((1, _PS, _PL), lambda i: (i, 0, 0)),
            pl.BlockSpec((1, 1, 128), lambda i: (i, 0, 0)),
        ],
        out_shape=[
            jax.ShapeDtypeStruct((_B, _PS, _PL), jnp.float32),
            jax.ShapeDtypeStruct((_B, 1, 128), jnp.float32),
        ],
    )(pt, targets, loc_t, conf_t)

    out_vec = _sc_select(jnp.reshape(ce, (_B, _P)),
                         jnp.reshape(part, (_B, 128)))
    return out_vec[0], out_vec[1]


# speedup vs baseline: 1.0296x; 1.0296x over previous
import functools

import jax
import jax.numpy as jnp
from jax import lax
from jax.experimental import pallas as pl
from jax.experimental.pallas import tpu as pltpu
from jax.experimental.pallas import tpu_sc as plsc

_THRESHOLD = 0.35
_VAR0, _VAR1 = 0.1, 0.2
_NEG_RATIO = 3
_B, _P, _NO = 16, 25600, 32
_PS, _PL = 200, 128


def _match_body(pt_ref, tgt_ref, loc_ref, conf_ref, ce_ref, part_ref):
    cx, cy, w, h = pt_ref[0], pt_ref[1], pt_ref[2], pt_ref[3]
    x0, y0 = cx - 0.5 * w, cy - 0.5 * h
    x1, y1 = cx + 0.5 * w, cy + 0.5 * h
    area_b = w * h
    pid = (jax.lax.broadcasted_iota(jnp.int32, (_PS, _PL), 0) * _PL
           + jax.lax.broadcasted_iota(jnp.int32, (_PS, _PL), 1))

    bto = jnp.full((_PS, _PL), -1.0, jnp.float32)
    bti = jnp.zeros((_PS, _PL), jnp.int32)
    f_idx = jnp.full((_PS, _PL), -1, jnp.int32)

    for j in range(_NO):
        tx0 = tgt_ref[0, j, 0]
        ty0 = tgt_ref[0, j, 1]
        tx1 = tgt_ref[0, j, 2]
        ty1 = tgt_ref[0, j, 3]
        iw = jnp.maximum(jnp.minimum(tx1, x1) - jnp.maximum(tx0, x0), 0.0)
        ih = jnp.maximum(jnp.minimum(ty1, y1) - jnp.maximum(ty0, y0), 0.0)
        inter = iw * ih
        aa = (tx1 - tx0) * (ty1 - ty0)
        ov = inter / (aa + area_b - inter)
        better = ov > bto
        bti = jnp.where(better, j, bti)
        bto = jnp.where(better, ov, bto)
        bpo = jnp.max(ov)
        bpi = jnp.min(jnp.where(ov == bpo, pid, _P))
        f_idx = jnp.where(pid == bpi, j, f_idx)

    forced = f_idx >= 0
    bti = jnp.where(forced, f_idx, bti)
    bto = jnp.where(forced, 2.0, bto)
    pos = bto >= _THRESHOLD
    posf = pos.astype(jnp.float32)

    msx = jnp.zeros((_PS, _PL), jnp.float32)
    mdx = jnp.zeros((_PS, _PL), jnp.float32)
    msy = jnp.zeros((_PS, _PL), jnp.float32)
    mdy = jnp.zeros((_PS, _PL), jnp.float32)
    for j in range(_NO):
        eq = bti == j
        msx = jnp.where(eq, tgt_ref[0, j, 0] + tgt_ref[0, j, 2], msx)
        mdx = jnp.where(eq, tgt_ref[0, j, 2] - tgt_ref[0, j, 0], mdx)
        msy = jnp.where(eq, tgt_ref[0, j, 1] + tgt_ref[0, j, 3], msy)
        mdy = jnp.where(eq, tgt_ref[0, j, 3] - tgt_ref[0, j, 1], mdy)

    inv_w, inv_h = 1.0 / w, 1.0 / h
    g_cx = (0.5 * msx - cx) * ((1.0 / _VAR0) * inv_w)
    g_cy = (0.5 * msy - cy) * ((1.0 / _VAR0) * inv_h)
    g_w = jnp.log(mdx * inv_w) * (1.0 / _VAR1)
    g_h = jnp.log(mdy * inv_h) * (1.0 / _VAR1)

    def _sl1(d):
        ad = jnp.abs(d)
        return jnp.where(ad < 1.0, 0.5 * d * d, ad - 0.5)

    sl1 = (_sl1(loc_ref[0, 0] - g_cx) + _sl1(loc_ref[0, 1] - g_cy)
           + _sl1(loc_ref[0, 2] - g_w) + _sl1(loc_ref[0, 3] - g_h))
    sl1_sum = jnp.sum(sl1 * posf)
    npos = jnp.sum(posf)

    c0, c1 = conf_ref[0, 0], conf_ref[0, 1]
    dng = jnp.where(pos, c0 - c1, c1 - c0)
    ce = jnp.maximum(dng, 0.0) + jnp.log1p(jnp.exp(-jnp.abs(dng)))
    pce = jnp.sum(ce * posf)

    ce_ref[...] = jnp.where(pos, -1.0, ce)[None]

    li = jax.lax.broadcasted_iota(jnp.int32, (1, 1, 128), 2)
    part_ref[...] = (jnp.where(li == 0, sl1_sum, 0.0)
                     + jnp.where(li == 1, npos, 0.0)
                     + jnp.where(li == 2, pce, 0.0))


def _select_body(ce_ref, part_ref, out_loc_ref, out_conf_ref):
    ce = ce_ref[...]
    part = part_ref[...]
    sl1 = part[:, :, 0:1]
    nposf = part[:, :, 1:2]
    pce = part[:, :, 2:3]

    s_total = jnp.sum(nposf)
    k = jnp.minimum(jnp.minimum(_NEG_RATIO * nposf, float(_P - 1)),
                    float(_P) - nposf)

    def _rowsum(x):
        return jnp.sum(jnp.sum(x, axis=1, keepdims=True), axis=2,
                       keepdims=True)

    ci = jax.lax.bitcast_convert_type(ce, jnp.int32)
    lo = jnp.zeros((_B, 1, 1), jnp.int32)
    hi = jnp.max(jnp.max(ci, axis=1, keepdims=True), axis=2,
                 keepdims=True) + 1

    def body(_, carry):
        lo, hi = carry
        mid = lo + jax.lax.div(hi - lo, 2)
        cnt = _rowsum(jnp.where(ci >= mid, 1.0, 0.0))
        ok = cnt >= k
        return jnp.where(ok, mid, lo), jnp.where(ok, hi, mid)

    lo, hi = jax.lax.fori_loop(0, 31, body, (lo, hi))
    t = lo
    tf = jax.lax.bitcast_convert_type(t, jnp.float32)
    gt = ci > t
    cnt_gt = _rowsum(jnp.where(gt, 1.0, 0.0))
    sum_gt = _rowsum(jnp.where(gt, ce, 0.0))
    neg_sum = sum_gt + (k - cnt_gt) * tf

    total_ce = jnp.sum(pce) + jnp.sum(neg_sum)
    total_sel = s_total + jnp.sum(k)
    out_loc_ref[...] = jnp.reshape(jnp.sum(sl1) / (4.0 * s_total) / s_total,
                                   (1, 1))
    out_conf_ref[...] = jnp.reshape(total_ce / total_sel / s_total, (1, 1))


_L = 16
_NB1 = 512
_NB2 = 1024


def _sc_select_body(ce_hbm, part_hbm, out_hbm, row_v, part_v, cnt1, sum1,
                    cnt2, sum2, cc1, cs1, cc2, cs2, stat_v, out_v, shared,
                    dma_sem):
    r = lax.axis_index("s")
    li = lax.iota(jnp.int32, _L)
    ones = jnp.full((_L,), 1.0, jnp.float32)
    zeros = jnp.zeros((_L,), jnp.float32)

    cp_row = pltpu.make_async_copy(ce_hbm.at[r], row_v, dma_sem)
    cp_row.start()
    pltpu.sync_copy(part_hbm.at[r], part_v)

    def _zero1(i, _):
        for u in range(8):
            cnt1[pl.ds((i * 8 + u) * _L, _L)] = zeros
            sum1[pl.ds((i * 8 + u) * _L, _L)] = zeros
        return 0

    def _zero2(i, _):
        for u in range(8):
            cnt2[pl.ds((i * 8 + u) * _L, _L)] = zeros
            sum2[pl.ds((i * 8 + u) * _L, _L)] = zeros
        return 0

    lax.fori_loop(0, _NB1 // 8, _zero1, 0)
    lax.fori_loop(0, _NB2 // 8, _zero2, 0)
    cp_row.wait()

    pv = part_v[pl.ds(0, _L)]
    sl1 = pv[0]
    npos = pv[1]
    pce = pv[2]
    k = jnp.minimum(jnp.minimum(_NEG_RATIO * npos, float(_P - 1)),
                    float(_P) - npos)

    def _hist1(i, _):
        for u in range(8):
            v = row_v[pl.ds((i * 8 + u) * _L, _L)]
            bits = lax.bitcast_convert_type(v, jnp.int32)
            msk = v >= 0.0
            b1 = jnp.clip(lax.shift_right_arithmetic(bits, 22), 0, _NB1 - 1)
            idx = li * _NB1 + b1
            plsc.addupdate_scatter(cnt1, [idx], ones, mask=msk)
            plsc.addupdate_scatter(sum1, [idx], v, mask=msk)
        return 0

    lax.fori_loop(0, _P // _L // 8, _hist1, 0)

    def _compact(tab, out, nb):
        def body(i, _):
            acc = tab[pl.ds(i * _L, _L)]
            for l in range(1, _L):
                acc = acc + tab[pl.ds(l * nb + i * _L, _L)]
            out[pl.ds(i * _L, _L)] = acc
            return 0

        lax.fori_loop(0, nb // _L, body, 0)

    _compact(cnt1, cc1, _NB1)
    _compact(sum1, cs1, _NB1)

    def _scan(cnt_tab, sum_tab, nb, kk):
        def cond(c):
            b, rc, rs, done = c
            return jnp.logical_and(jnp.logical_not(done), b >= 0)

        def body(c):
            b, rc, rs, done = c
            cb = cnt_tab[pl.ds(b, _L)][0]
            sb = sum_tab[pl.ds(b, _L)][0]
            hit = rc + cb >= kk
            rc = jnp.where(hit, rc, rc + cb)
            rs = jnp.where(hit, rs, rs + sb)
            b = jnp.where(hit, b, b - 1)
            return b, rc, rs, hit

        b, rc, rs, _ = lax.while_loop(
            cond, body, (jnp.int32(nb - 1), 0.0, 0.0, False))
        return b, rc, rs

    b1s, cnt_gt1, sum_gt1 = _scan(cc1, cs1, _NB1, k)
    k2 = k - cnt_gt1

    def _hist2(i, _):
        for u in range(8):
            v = row_v[pl.ds((i * 8 + u) * _L, _L)]
            bits = lax.bitcast_convert_type(v, jnp.int32)
            b1 = jnp.clip(lax.shift_right_arithmetic(bits, 22), 0, _NB1 - 1)
            msk = jnp.logical_and(v >= 0.0, b1 == b1s)
            b2 = jnp.bitwise_and(lax.shift_right_arithmetic(bits, 12), _NB2 - 1)
            idx = li * _NB2 + b2
            plsc.addupdate_scatter(cnt2, [idx], ones, mask=msk)
            plsc.addupdate_scatter(sum2, [idx], v, mask=msk)
        return 0

    lax.fori_loop(0, _P // _L // 8, _hist2, 0)

    _compact(cnt2, cc2, _NB2)
    _compact(sum2, cs2, _NB2)

    b2s, cnt_gt2, sum_gt2 = _scan(cc2, cs2, _NB2, k2)
    cnt_at_v = cc2[pl.ds(b2s, _L)]
    sum_at_v = cs2[pl.ds(b2s, _L)]
    mean_at = (sum_at_v / cnt_at_v)[0]
    neg_sum = sum_gt1 + sum_gt2 + (k2 - cnt_gt2) * mean_at

    stat_v[...] = (jnp.where(li == 0, neg_sum, 0.0)
                   + jnp.where(li == 1, k, 0.0)
                   + jnp.where(li == 2, npos, 0.0)
                   + jnp.where(li == 3, sl1, 0.0)
                   + jnp.where(li == 4, pce, 0.0))
    pltpu.sync_copy(stat_v, shared.at[pl.ds(r * _L, _L)])
    plsc.subcore_barrier()

    @pl.when(r == 0)
    def _combine():
        def _acc(i, acc):
            pltpu.sync_copy(shared.at[pl.ds(i * _L, _L)], stat_v)
            return acc + stat_v[...]

        acc = lax.fori_loop(0, _B, _acc, zeros)
        neg_tot = acc[0]
        k_tot = acc[1]
        s_tot = acc[2]
        sl1_tot = acc[3]
        pce_tot = acc[4]
        num_v = (jnp.where(li == 0, sl1_tot, 0.0)
                 + jnp.where(li == 1, pce_tot + neg_tot, 0.0))
        den_v = (jnp.where(li == 0, 4.0 * s_tot * s_tot, 1.0)
                 + jnp.where(li == 1, (s_tot + k_tot) * s_tot - 1.0, 0.0))
        out_v[...] = num_v / den_v
        pltpu.sync_copy(out_v, out_hbm)


def _sc_select(ce16, part2):
    mesh = plsc.VectorSubcoreMesh(core_axis_name="c", subcore_axis_name="s",
                                  num_cores=1)
    f = functools.partial(
        pl.kernel, mesh=mesh,
        compiler_params=pltpu.CompilerParams(needs_layout_passes=False),
        out_type=jax.ShapeDtypeStruct((_L,), jnp.float32),
        scratch_types=[
            pltpu.VMEM((_P,), jnp.float32),
            pltpu.VMEM((128,), jnp.float32),
            pltpu.VMEM((_L * _NB1,), jnp.float32),
            pltpu.VMEM((_L * _NB1,), jnp.float32),
            pltpu.VMEM((_L * _NB2,), jnp.float32),
            pltpu.VMEM((_L * _NB2,), jnp.float32),
            pltpu.VMEM((_NB1 + _L,), jnp.float32),
            pltpu.VMEM((_NB1 + _L,), jnp.float32),
            pltpu.VMEM((_NB2 + _L,), jnp.float32),
            pltpu.VMEM((_NB2 + _L,), jnp.float32),
            pltpu.VMEM((_L,), jnp.float32),
            pltpu.VMEM((_L,), jnp.float32),
            pltpu.VMEM_SHARED((_B * _L,), jnp.float32),
            pltpu.SemaphoreType.DMA,
        ],
    )(_sc_select_body)
    return f(ce16, part2)


def kernel(loc_data, conf_data, priors, targets):
    pt = priors.T.reshape(4, _PS, _PL)
    loc_t = jnp.transpose(loc_data, (0, 2, 1)).reshape(_B, 4, _PS, _PL)
    conf_t = jnp.transpose(conf_data, (0, 2, 1)).reshape(_B, 2, _PS, _PL)

    ce, part = pl.pallas_call(
        _match_body,
        grid=(_B,),
        in_specs=[
            pl.BlockSpec((4, _PS, _PL), lambda i: (0, 0, 0)),
            pl.BlockSpec((1, _NO, 5), lambda i: (i, 0, 0),
                         memory_space=pltpu.SMEM),
            pl.BlockSpec((1, 4, _PS, _PL), lambda i: (i, 0, 0, 0)),
            pl.BlockSpec((1, 2, _PS, _PL), lambda i: (i, 0, 0, 0)),
        ],
        out_specs=[
            pl.BlockSpec((1, _PS, _PL), lambda i: (i, 0, 0)),
            pl.BlockSpec((1, 1, 128), lambda i: (i, 0, 0)),
        ],
        out_shape=[
            jax.ShapeDtypeStruct((_B, _PS, _PL), jnp.float32),
            jax.ShapeDtypeStruct((_B, 1, 128), jnp.float32),
        ],
    )(pt, targets, loc_t, conf_t)

    out_vec = _sc_select(jnp.reshape(ce, (_B, _P)),
                         jnp.reshape(part, (_B, 128)))
    return out_vec[0], out_vec[1]


# speedup vs baseline: 1.1252x; 1.0929x over previous
import functools

import jax
import jax.numpy as jnp
from jax import lax
from jax.experimental import pallas as pl
from jax.experimental.pallas import tpu as pltpu
from jax.experimental.pallas import tpu_sc as plsc

_THRESHOLD = 0.35
_VAR0, _VAR1 = 0.1, 0.2
_NEG_RATIO = 3
_B, _P, _NO = 16, 25600, 32
_PS, _PL = 200, 128


def _match_body(pt_ref, tgt_ref, loc_ref, conf_ref, ce_ref, part_ref):
    cx, cy, w, h = pt_ref[0], pt_ref[1], pt_ref[2], pt_ref[3]
    x0, y0 = cx - 0.5 * w, cy - 0.5 * h
    x1, y1 = cx + 0.5 * w, cy + 0.5 * h
    area_b = w * h
    pid = (jax.lax.broadcasted_iota(jnp.int32, (_PS, _PL), 0) * _PL
           + jax.lax.broadcasted_iota(jnp.int32, (_PS, _PL), 1))

    bto = jnp.full((_PS, _PL), -1.0, jnp.float32)
    bti = jnp.zeros((_PS, _PL), jnp.int32)
    f_idx = jnp.full((_PS, _PL), -1, jnp.int32)

    for j in range(_NO):
        tx0 = tgt_ref[0, j, 0]
        ty0 = tgt_ref[0, j, 1]
        tx1 = tgt_ref[0, j, 2]
        ty1 = tgt_ref[0, j, 3]
        iw = jnp.maximum(jnp.minimum(tx1, x1) - jnp.maximum(tx0, x0), 0.0)
        ih = jnp.maximum(jnp.minimum(ty1, y1) - jnp.maximum(ty0, y0), 0.0)
        inter = iw * ih
        aa = (tx1 - tx0) * (ty1 - ty0)
        ov = inter / (aa + area_b - inter)
        better = ov > bto
        bti = jnp.where(better, j, bti)
        bto = jnp.where(better, ov, bto)
        bpo = jnp.max(ov)
        bpi = jnp.min(jnp.where(ov == bpo, pid, _P))
        f_idx = jnp.where(pid == bpi, j, f_idx)

    forced = f_idx >= 0
    bti = jnp.where(forced, f_idx, bti)
    bto = jnp.where(forced, 2.0, bto)
    pos = bto >= _THRESHOLD
    posf = pos.astype(jnp.float32)

    msx = jnp.zeros((_PS, _PL), jnp.float32)
    mdx = jnp.zeros((_PS, _PL), jnp.float32)
    msy = jnp.zeros((_PS, _PL), jnp.float32)
    mdy = jnp.zeros((_PS, _PL), jnp.float32)
    for j in range(_NO):
        eq = bti == j
        msx = jnp.where(eq, tgt_ref[0, j, 0] + tgt_ref[0, j, 2], msx)
        mdx = jnp.where(eq, tgt_ref[0, j, 2] - tgt_ref[0, j, 0], mdx)
        msy = jnp.where(eq, tgt_ref[0, j, 1] + tgt_ref[0, j, 3], msy)
        mdy = jnp.where(eq, tgt_ref[0, j, 3] - tgt_ref[0, j, 1], mdy)

    inv_w, inv_h = 1.0 / w, 1.0 / h
    g_cx = (0.5 * msx - cx) * ((1.0 / _VAR0) * inv_w)
    g_cy = (0.5 * msy - cy) * ((1.0 / _VAR0) * inv_h)
    g_w = jnp.log(mdx * inv_w) * (1.0 / _VAR1)
    g_h = jnp.log(mdy * inv_h) * (1.0 / _VAR1)

    def _sl1(d):
        ad = jnp.abs(d)
        return jnp.where(ad < 1.0, 0.5 * d * d, ad - 0.5)

    sl1 = (_sl1(loc_ref[0, 0] - g_cx) + _sl1(loc_ref[0, 1] - g_cy)
           + _sl1(loc_ref[0, 2] - g_w) + _sl1(loc_ref[0, 3] - g_h))
    sl1_sum = jnp.sum(sl1 * posf)
    npos = jnp.sum(posf)

    c0, c1 = conf_ref[0, 0], conf_ref[0, 1]
    dng = jnp.where(pos, c0 - c1, c1 - c0)
    ce = jnp.maximum(dng, 0.0) + jnp.log1p(jnp.exp(-jnp.abs(dng)))
    pce = jnp.sum(ce * posf)

    ce_ref[...] = jnp.where(pos, -1.0, ce)[None]

    li = jax.lax.broadcasted_iota(jnp.int32, (1, 1, 128), 2)
    part_ref[...] = (jnp.where(li == 0, sl1_sum, 0.0)
                     + jnp.where(li == 1, npos, 0.0)
                     + jnp.where(li == 2, pce, 0.0))


def _select_body(ce_ref, part_ref, out_loc_ref, out_conf_ref):
    ce = ce_ref[...]
    part = part_ref[...]
    sl1 = part[:, :, 0:1]
    nposf = part[:, :, 1:2]
    pce = part[:, :, 2:3]

    s_total = jnp.sum(nposf)
    k = jnp.minimum(jnp.minimum(_NEG_RATIO * nposf, float(_P - 1)),
                    float(_P) - nposf)

    def _rowsum(x):
        return jnp.sum(jnp.sum(x, axis=1, keepdims=True), axis=2,
                       keepdims=True)

    ci = jax.lax.bitcast_convert_type(ce, jnp.int32)
    lo = jnp.zeros((_B, 1, 1), jnp.int32)
    hi = jnp.max(jnp.max(ci, axis=1, keepdims=True), axis=2,
                 keepdims=True) + 1

    def body(_, carry):
        lo, hi = carry
        mid = lo + jax.lax.div(hi - lo, 2)
        cnt = _rowsum(jnp.where(ci >= mid, 1.0, 0.0))
        ok = cnt >= k
        return jnp.where(ok, mid, lo), jnp.where(ok, hi, mid)

    lo, hi = jax.lax.fori_loop(0, 31, body, (lo, hi))
    t = lo
    tf = jax.lax.bitcast_convert_type(t, jnp.float32)
    gt = ci > t
    cnt_gt = _rowsum(jnp.where(gt, 1.0, 0.0))
    sum_gt = _rowsum(jnp.where(gt, ce, 0.0))
    neg_sum = sum_gt + (k - cnt_gt) * tf

    total_ce = jnp.sum(pce) + jnp.sum(neg_sum)
    total_sel = s_total + jnp.sum(k)
    out_loc_ref[...] = jnp.reshape(jnp.sum(sl1) / (4.0 * s_total) / s_total,
                                   (1, 1))
    out_conf_ref[...] = jnp.reshape(total_ce / total_sel / s_total, (1, 1))


_L = 16
_NB1 = 512
_NB2 = 1024


def _sc_select_body(ce_hbm, part_hbm, out_hbm, row_v, part_v, cnt1, sum1,
                    cnt2, sum2, cc1, cs1, cc2, cs2, stat_v, out_v, shared,
                    dma_sem):
    r = lax.axis_index("s")
    li = lax.iota(jnp.int32, _L)
    ones = jnp.full((_L,), 1.0, jnp.float32)
    zeros = jnp.zeros((_L,), jnp.float32)

    cp_row = pltpu.make_async_copy(ce_hbm.at[r], row_v, dma_sem)
    cp_row.start()
    pltpu.sync_copy(part_hbm.at[r], part_v)

    def _zero1(i, _):
        for u in range(8):
            cnt1[pl.ds((i * 8 + u) * _L, _L)] = zeros
            sum1[pl.ds((i * 8 + u) * _L, _L)] = zeros
        return 0

    def _zero2(i, _):
        for u in range(8):
            cnt2[pl.ds((i * 8 + u) * _L, _L)] = zeros
            sum2[pl.ds((i * 8 + u) * _L, _L)] = zeros
        return 0

    lax.fori_loop(0, _NB1 // 8, _zero1, 0)
    lax.fori_loop(0, _NB2 // 8, _zero2, 0)
    cp_row.wait()

    pv = part_v[pl.ds(0, _L)]
    sl1 = pv[0]
    npos = pv[1]
    pce = pv[2]
    k = jnp.minimum(jnp.minimum(_NEG_RATIO * npos, float(_P - 1)),
                    float(_P) - npos)

    def _hist1(i, _):
        for u in range(8):
            v = row_v[pl.ds((i * 8 + u) * _L, _L)]
            bits = lax.bitcast_convert_type(v, jnp.int32)
            msk = v >= 0.0
            b1 = jnp.clip(lax.shift_right_arithmetic(bits, 22), 0, _NB1 - 1)
            idx = li * _NB1 + b1
            plsc.addupdate_scatter(cnt1, [idx], ones, mask=msk)
            plsc.addupdate_scatter(sum1, [idx], v, mask=msk)
        return 0

    lax.fori_loop(0, _P // _L // 8, _hist1, 0)

    def _compact(tab, out, nb):
        def body(i, _):
            acc = tab[pl.ds(i * _L, _L)]
            for l in range(1, _L):
                acc = acc + tab[pl.ds(l * nb + i * _L, _L)]
            out[pl.ds(i * _L, _L)] = acc
            return 0

        lax.fori_loop(0, nb // _L, body, 0)

    _compact(cnt1, cc1, _NB1)
    _compact(sum1, cs1, _NB1)

    def _sfx(v):
        for sh in (1, 2, 4, 8):
            idx = jnp.minimum(li + sh, _L - 1)
            shifted = jnp.where(li < _L - sh,
                                v.at[idx].get(mode="promise_in_bounds"),
                                0.0)
            v = v + shifted
        return v

    def _scan(cnt_tab, sum_tab, nb, kk):
        def cond(c):
            i, rc, rs, done = c
            return jnp.logical_and(jnp.logical_not(done), i >= 0)

        def body(c):
            i, rc, rs, done = c
            tc = _sfx(cnt_tab[pl.ds(i * _L, _L)])[0]
            ts = _sfx(sum_tab[pl.ds(i * _L, _L)])[0]
            hit = rc + tc >= kk
            rc = jnp.where(hit, rc, rc + tc)
            rs = jnp.where(hit, rs, rs + ts)
            i = jnp.where(hit, i, i - 1)
            return i, rc, rs, hit

        i, rc, rs, _ = lax.while_loop(
            cond, body, (jnp.int32(nb // _L - 1), 0.0, 0.0, False))
        rcum = _sfx(cnt_tab[pl.ds(i * _L, _L)]) + rc
        rsum = _sfx(sum_tab[pl.ds(i * _L, _L)]) + rs
        mask = rcum >= kk
        bl = plsc.all_reduce_population_count(mask)[0] - 1
        b = i * _L + bl
        nxt = jnp.full((_L,), jnp.minimum(bl + 1, _L - 1), jnp.int32)
        cnt_gt = jnp.where(bl == _L - 1, rc,
                           rcum.at[nxt].get(mode="promise_in_bounds")[0])
        sum_gt = jnp.where(bl == _L - 1, rs,
                           rsum.at[nxt].get(mode="promise_in_bounds")[0])
        return b, cnt_gt, sum_gt

    b1s, cnt_gt1, sum_gt1 = _scan(cc1, cs1, _NB1, k)
    k2 = k - cnt_gt1

    def _hist2(i, _):
        for u in range(8):
            v = row_v[pl.ds((i * 8 + u) * _L, _L)]
            bits = lax.bitcast_convert_type(v, jnp.int32)
            b1 = jnp.clip(lax.shift_right_arithmetic(bits, 22), 0, _NB1 - 1)
            msk = jnp.logical_and(v >= 0.0, b1 == b1s)
            b2 = jnp.bitwise_and(lax.shift_right_arithmetic(bits, 12), _NB2 - 1)
            idx = li * _NB2 + b2
            plsc.addupdate_scatter(cnt2, [idx], ones, mask=msk)
            plsc.addupdate_scatter(sum2, [idx], v, mask=msk)
        return 0

    lax.fori_loop(0, _P // _L // 8, _hist2, 0)

    _compact(cnt2, cc2, _NB2)
    _compact(sum2, cs2, _NB2)

    b2s, cnt_gt2, sum_gt2 = _scan(cc2, cs2, _NB2, k2)
    cnt_at_v = cc2[pl.ds(b2s, _L)]
    sum_at_v = cs2[pl.ds(b2s, _L)]
    mean_at = (sum_at_v / cnt_at_v)[0]
    neg_sum = sum_gt1 + sum_gt2 + (k2 - cnt_gt2) * mean_at

    stat_v[...] = (jnp.where(li == 0, neg_sum, 0.0)
                   + jnp.where(li == 1, k, 0.0)
                   + jnp.where(li == 2, npos, 0.0)
                   + jnp.where(li == 3, sl1, 0.0)
                   + jnp.where(li == 4, pce, 0.0))
    pltpu.sync_copy(stat_v, shared.at[pl.ds(r * _L, _L)])
    plsc.subcore_barrier()

    @pl.when(r == 0)
    def _combine():
        def _acc(i, acc):
            pltpu.sync_copy(shared.at[pl.ds(i * _L, _L)], stat_v)
            return acc + stat_v[...]

        acc = lax.fori_loop(0, _B, _acc, zeros)
        neg_tot = acc[0]
        k_tot = acc[1]
        s_tot = acc[2]
        sl1_tot = acc[3]
        pce_tot = acc[4]
        num_v = (jnp.where(li == 0, sl1_tot, 0.0)
                 + jnp.where(li == 1, pce_tot + neg_tot, 0.0))
        den_v = (jnp.where(li == 0, 4.0 * s_tot * s_tot, 1.0)
                 + jnp.where(li == 1, (s_tot + k_tot) * s_tot - 1.0, 0.0))
        out_v[...] = num_v / den_v
        pltpu.sync_copy(out_v, out_hbm)


def _sc_select(ce16, part2):
    mesh = plsc.VectorSubcoreMesh(core_axis_name="c", subcore_axis_name="s",
                                  num_cores=1)
    f = functools.partial(
        pl.kernel, mesh=mesh,
        compiler_params=pltpu.CompilerParams(needs_layout_passes=False),
        out_type=jax.ShapeDtypeStruct((_L,), jnp.float32),
        scratch_types=[
            pltpu.VMEM((_P,), jnp.float32),
            pltpu.VMEM((128,), jnp.float32),
            pltpu.VMEM((_L * _NB1,), jnp.float32),
            pltpu.VMEM((_L * _NB1,), jnp.float32),
            pltpu.VMEM((_L * _NB2,), jnp.float32),
            pltpu.VMEM((_L * _NB2,), jnp.float32),
            pltpu.VMEM((_NB1 + _L,), jnp.float32),
            pltpu.VMEM((_NB1 + _L,), jnp.float32),
            pltpu.VMEM((_NB2 + _L,), jnp.float32),
            pltpu.VMEM((_NB2 + _L,), jnp.float32),
            pltpu.VMEM((_L,), jnp.float32),
            pltpu.VMEM((_L,), jnp.float32),
            pltpu.VMEM_SHARED((_B * _L,), jnp.float32),
            pltpu.SemaphoreType.DMA,
        ],
    )(_sc_select_body)
    return f(ce16, part2)


def kernel(loc_data, conf_data, priors, targets):
    pt = priors.T.reshape(4, _PS, _PL)
    loc_t = jnp.transpose(loc_data, (0, 2, 1)).reshape(_B, 4, _PS, _PL)
    conf_t = jnp.transpose(conf_data, (0, 2, 1)).reshape(_B, 2, _PS, _PL)

    ce, part = pl.pallas_call(
        _match_body,
        grid=(_B,),
        in_specs=[
            pl.BlockSpec((4, _PS, _PL), lambda i: (0, 0, 0)),
            pl.BlockSpec((1, _NO, 5), lambda i: (i, 0, 0),
                         memory_space=pltpu.SMEM),
            pl.BlockSpec((1, 4, _PS, _PL), lambda i: (i, 0, 0, 0)),
            pl.BlockSpec((1, 2, _PS, _PL), lambda i: (i, 0, 0, 0)),
        ],
        out_specs=[
            pl.BlockSpec((1, _PS, _PL), lambda i: (i, 0, 0)),
            pl.BlockSpec((1, 1, 128), lambda i: (i, 0, 0)),
        ],
        out_shape=[
            jax.ShapeDtypeStruct((_B, _PS, _PL), jnp.float32),
            jax.ShapeDtypeStruct((_B, 1, 128), jnp.float32),
        ],
    )(pt, targets, loc_t, conf_t)

    out_vec = _sc_select(jnp.reshape(ce, (_B, _P)),
                         jnp.reshape(part, (_B, 128)))
    return out_vec[0], out_vec[1]
